# Initial kernel scaffold; baseline (speedup 1.0000x reference)
import jax
import jax.numpy as jnp
from jax.experimental import pallas as pl


def kernel(x_entropy_p16, x_entropy_p8):
    raise NotImplementedError("write your pallas kernel here")


# speedup vs baseline: 8.0054x; 8.0054x over previous
import functools

import jax
import jax.numpy as jnp
from jax import lax
from jax.experimental import pallas as pl
from jax.experimental.pallas import tpu as pltpu

_COARSE = 0.3
_MEDIUM = 0.4
_N16 = 256 * 32 * 32
_N8 = 256 * 64 * 64
_K1 = round(_N16 * _COARSE)
_K2 = round(4 * _N16 * _COARSE + _N8 * _MEDIUM)


def _count_less(arrs, t):
    s = jnp.int32(0)
    for a in arrs:
        s = s + jnp.sum((a < t).astype(jnp.int32))
    return s


def _bisect_kth(arrs, k):
    lo = jnp.int32(0)
    for r in range(15):
        w = 1 << (28 - 2 * r)
        t1 = lo + w
        t2 = lo + 2 * w
        t3 = lo + 3 * w
        s1 = _count_less(arrs, t1)
        s2 = _count_less(arrs, t2)
        s3 = _count_less(arrs, t3)
        lo = jnp.where(k <= s1, lo,
                       jnp.where(k <= s2, t1,
                                 jnp.where(k <= s3, t2, t3)))
    return lo


def _select_body(x16_ref, x8_ref, oc_ref, om_ref):
    v16 = lax.bitcast_convert_type(x16_ref[...], jnp.int32)
    c_bits = _bisect_kth([v16], _K1)

    m16 = (v16 < c_bits).astype(jnp.bfloat16)

    x8v = x8_ref[...]
    row = lax.broadcasted_iota(jnp.int32, (128, 128), 0)
    col = lax.broadcasted_iota(jnp.int32, (128, 128), 1)
    masked = []
    for r in range(4):
        p_r = (row == (32 * r + (col % 64) // 2)).astype(jnp.bfloat16)
        mcols = jnp.dot(m16, p_r, preferred_element_type=jnp.float32)
        vals = jnp.where(mcols > 0.5, 0.0, x8v[:, 128 * r:128 * (r + 1)])
        masked.append(lax.bitcast_convert_type(vals, jnp.int32))
    m_bits = _bisect_kth(masked, _K2)

    oc_ref[0, 0] = c_bits
    om_ref[0, 0] = m_bits


def _expand(rows, cols):
    return (rows[:, None] == cols[None, :]).astype(jnp.bfloat16)


def _gates_body(x16_ref, x8_ref, ct_ref, mt_ref, gc_ref, gm_ref, gf_ref,
                gl_ref):
    cthr = ct_ref[0, 0]
    mthr = mt_ref[0, 0]
    x16 = x16_ref[...].reshape(256, 32)
    x8 = x8_ref[...].reshape(512, 64)

    i1024 = lax.iota(jnp.int32, 1024)
    i512 = lax.iota(jnp.int32, 512)
    i384 = lax.iota(jnp.int32, 384)
    i256 = lax.iota(jnp.int32, 256)
    i128 = lax.iota(jnp.int32, 128)
    i64 = lax.iota(jnp.int32, 64)
    i32 = lax.iota(jnp.int32, 32)

    gcf = (x16 < cthr).astype(jnp.bfloat16)
    gc_ref[...] = gcf.astype(jnp.int32).reshape(8, 32, 32)

    r2m = _expand(32 * (i512 // 64) + (i512 % 64) // 2, i256)
    c2c = _expand(i32, i64 // 2)
    gc2 = jnp.dot(jnp.dot(r2m, gcf, preferred_element_type=jnp.float32)
                  .astype(jnp.bfloat16), c2c,
                  preferred_element_type=jnp.float32)

    gmf = ((x8 < mthr) & (gc2 < 0.5)).astype(jnp.bfloat16)
    gm_ref[...] = gmf.astype(jnp.int32).reshape(8, 64, 64)

    r4r = _expand(32 * (i1024 // 128) + (i1024 % 128) // 4, i256)
    c4c = _expand(i32, i128 // 4)
    gc4 = jnp.dot(jnp.dot(r4r, gcf, preferred_element_type=jnp.float32)
                  .astype(jnp.bfloat16), c4c,
                  preferred_element_type=jnp.float32)

    r2r = _expand(64 * (i1024 // 128) + (i1024 % 128) // 2, i512)
    c2f = _expand(i64, i128 // 2)
    gm2 = jnp.dot(jnp.dot(r2r, gmf, preferred_element_type=jnp.float32)
                  .astype(jnp.bfloat16), c2f,
                  preferred_element_type=jnp.float32)

    gff = 1.0 - gc4 - gm2
    gf_ref[...] = gff.astype(jnp.int32).reshape(8, 128, 128)

    cls = (2.0 - 2.0 * gc4 - gm2).astype(jnp.bfloat16)
    e3 = _expand(i128, i384 // 3)
    cls3 = jnp.dot(cls, e3, preferred_element_type=jnp.float32)
    mod3 = (lax.broadcasted_iota(jnp.int32, (1024, 384), 1) % 3).astype(
        jnp.float32)
    gl_ref[...] = (cls3 == mod3).astype(jnp.int32).reshape(8, 128, 384)


@jax.jit
def kernel(x_entropy_p16, x_entropy_p8):
    x16f = x_entropy_p16.reshape(2048, 128)
    x8f = x_entropy_p8.reshape(2048, 512)

    c_bits, m_bits = pl.pallas_call(
        _select_body,
        out_specs=(pl.BlockSpec(memory_space=pltpu.SMEM),
                   pl.BlockSpec(memory_space=pltpu.SMEM)),
        out_shape=(jax.ShapeDtypeStruct((1, 1), jnp.int32),
                   jax.ShapeDtypeStruct((1, 1), jnp.int32)),
    )(x16f, x8f)
    cthr = lax.bitcast_convert_type(c_bits, jnp.float32)
    mthr = lax.bitcast_convert_type(m_bits, jnp.float32)

    grid = 32
    gc, gm, gf, gl = pl.pallas_call(
        _gates_body,
        grid=(grid,),
        in_specs=[
            pl.BlockSpec((8, 32, 32), lambda b: (b, 0, 0)),
            pl.BlockSpec((8, 64, 64), lambda b: (b, 0, 0)),
            pl.BlockSpec((1, 1), lambda b: (0, 0)),
            pl.BlockSpec((1, 1), lambda b: (0, 0)),
        ],
        out_specs=[
            pl.BlockSpec((8, 32, 32), lambda b: (b, 0, 0)),
            pl.BlockSpec((8, 64, 64), lambda b: (b, 0, 0)),
            pl.BlockSpec((8, 128, 128), lambda b: (b, 0, 0)),
            pl.BlockSpec((8, 128, 384), lambda b: (b, 0, 0)),
        ],
        out_shape=(
            jax.ShapeDtypeStruct((256, 32, 32), jnp.int32),
            jax.ShapeDtypeStruct((256, 64, 64), jnp.int32),
            jax.ShapeDtypeStruct((256, 128, 128), jnp.int32),
            jax.ShapeDtypeStruct((256, 128, 384), jnp.int32),
        ),
    )(x_entropy_p16, x_entropy_p8, cthr, mthr)
    return gc, gm, gf, gl.reshape(256, 128, 128, 3)


# speedup vs baseline: 14.7314x; 1.8402x over previous
import functools

import jax
import jax.numpy as jnp
from jax import lax
from jax.experimental import pallas as pl
from jax.experimental.pallas import tpu as pltpu

_COARSE = 0.3
_MEDIUM = 0.4
_N16 = 256 * 32 * 32
_N8 = 256 * 64 * 64
_K1 = round(_N16 * _COARSE)
_K2 = round(4 * _N16 * _COARSE + _N8 * _MEDIUM)


def _count_less(arrs, t):
    s = jnp.int32(0)
    for a in arrs:
        s = s + jnp.sum((a < t).astype(jnp.int32))
    return s


def _bisect_kth(arrs, k):
    lo = jnp.int32(0)
    for r in range(15):
        w = 1 << (28 - 2 * r)
        t1 = lo + w
        t2 = lo + 2 * w
        t3 = lo + 3 * w
        s1 = _count_less(arrs, t1)
        s2 = _count_less(arrs, t2)
        s3 = _count_less(arrs, t3)
        lo = jnp.where(k <= s1, lo,
                       jnp.where(k <= s2, t1,
                                 jnp.where(k <= s3, t2, t3)))
    return lo


def _select_body(x16_ref, x8_ref, oc_ref, om_ref):
    v16 = lax.bitcast_convert_type(x16_ref[...], jnp.int32)
    c_bits = _bisect_kth([v16], _K1)

    m16 = (v16 < c_bits).astype(jnp.bfloat16)

    x8v = x8_ref[...]
    row = lax.broadcasted_iota(jnp.int32, (128, 128), 0)
    col = lax.broadcasted_iota(jnp.int32, (128, 128), 1)
    masked = []
    for r in range(4):
        p_r = (row == (32 * r + (col % 64) // 2)).astype(jnp.bfloat16)
        mcols = jnp.dot(m16, p_r, preferred_element_type=jnp.float32)
        vals = jnp.where(mcols > 0.5, 0.0, x8v[:, 128 * r:128 * (r + 1)])
        masked.append(lax.bitcast_convert_type(vals, jnp.int32))
    m_bits = _bisect_kth(masked, _K2)

    oc_ref[0, 0] = c_bits
    om_ref[0, 0] = m_bits


def _expand(rows, cols):
    return (rows[:, None] == cols[None, :]).astype(jnp.bfloat16)


def _gates_body(x16_ref, x8_ref, ct_ref, mt_ref, gc_ref, gm_ref, gf_ref,
                gl_ref):
    cthr = ct_ref[0, 0]
    mthr = mt_ref[0, 0]
    x16 = x16_ref[...].reshape(256, 32)
    x8 = x8_ref[...].reshape(512, 64)

    i1024 = lax.iota(jnp.int32, 1024)
    i512 = lax.iota(jnp.int32, 512)
    i384 = lax.iota(jnp.int32, 384)
    i256 = lax.iota(jnp.int32, 256)
    i128 = lax.iota(jnp.int32, 128)
    i64 = lax.iota(jnp.int32, 64)
    i32 = lax.iota(jnp.int32, 32)

    gcf = (x16 < cthr).astype(jnp.bfloat16)
    gc_ref[...] = gcf.astype(jnp.int32).reshape(8, 32, 32)

    r2m = _expand(32 * (i512 // 64) + (i512 % 64) // 2, i256)
    c2c = _expand(i32, i64 // 2)
    gc2 = jnp.dot(jnp.dot(r2m, gcf, preferred_element_type=jnp.float32)
                  .astype(jnp.bfloat16), c2c,
                  preferred_element_type=jnp.float32)

    gmf = ((x8 < mthr) & (gc2 < 0.5)).astype(jnp.bfloat16)
    gm_ref[...] = gmf.astype(jnp.int32).reshape(8, 64, 64)

    r4r = _expand(32 * (i1024 // 128) + (i1024 % 128) // 4, i256)
    c4c = _expand(i32, i128 // 4)
    gc4 = jnp.dot(jnp.dot(r4r, gcf, preferred_element_type=jnp.float32)
                  .astype(jnp.bfloat16), c4c,
                  preferred_element_type=jnp.float32)

    r2r = _expand(64 * (i1024 // 128) + (i1024 % 128) // 2, i512)
    c2f = _expand(i64, i128 // 2)
    gm2 = jnp.dot(jnp.dot(r2r, gmf, preferred_element_type=jnp.float32)
                  .astype(jnp.bfloat16), c2f,
                  preferred_element_type=jnp.float32)

    gff = 1.0 - gc4 - gm2
    gf_ref[...] = gff.astype(jnp.int32).reshape(8, 128, 128)

    gl_ref[:, 0] = gc4.astype(jnp.int32).reshape(8, 128, 128)
    gl_ref[:, 1] = gm2.astype(jnp.int32).reshape(8, 128, 128)
    gl_ref[:, 2] = gff.astype(jnp.int32).reshape(8, 128, 128)


@jax.jit
def kernel(x_entropy_p16, x_entropy_p8):
    x16f = x_entropy_p16.reshape(2048, 128)
    x8f = x_entropy_p8.reshape(2048, 512)

    c_bits, m_bits = pl.pallas_call(
        _select_body,
        out_specs=(pl.BlockSpec(memory_space=pltpu.SMEM),
                   pl.BlockSpec(memory_space=pltpu.SMEM)),
        out_shape=(jax.ShapeDtypeStruct((1, 1), jnp.int32),
                   jax.ShapeDtypeStruct((1, 1), jnp.int32)),
    )(x16f, x8f)
    cthr = lax.bitcast_convert_type(c_bits, jnp.float32)
    mthr = lax.bitcast_convert_type(m_bits, jnp.float32)

    grid = 32
    gc, gm, gf, gl = pl.pallas_call(
        _gates_body,
        grid=(grid,),
        in_specs=[
            pl.BlockSpec((8, 32, 32), lambda b: (b, 0, 0)),
            pl.BlockSpec((8, 64, 64), lambda b: (b, 0, 0)),
            pl.BlockSpec((1, 1), lambda b: (0, 0)),
            pl.BlockSpec((1, 1), lambda b: (0, 0)),
        ],
        out_specs=[
            pl.BlockSpec((8, 32, 32), lambda b: (b, 0, 0)),
            pl.BlockSpec((8, 64, 64), lambda b: (b, 0, 0)),
            pl.BlockSpec((8, 128, 128), lambda b: (b, 0, 0)),
            pl.BlockSpec((8, 3, 128, 128), lambda b: (b, 0, 0, 0)),
        ],
        out_shape=(
            jax.ShapeDtypeStruct((256, 32, 32), jnp.int32),
            jax.ShapeDtypeStruct((256, 64, 64), jnp.int32),
            jax.ShapeDtypeStruct((256, 128, 128), jnp.int32),
            jax.ShapeDtypeStruct((256, 3, 128, 128), jnp.int32),
        ),
    )(x_entropy_p16, x_entropy_p8, cthr, mthr)
    return gc, gm, gf, gl.transpose(0, 2, 3, 1)


# speedup vs baseline: 16.1764x; 1.0981x over previous
import functools

import jax
import jax.numpy as jnp
from jax import lax
from jax.experimental import pallas as pl
from jax.experimental.pallas import tpu as pltpu

_COARSE = 0.3
_MEDIUM = 0.4
_N16 = 256 * 32 * 32
_N8 = 256 * 64 * 64
_K1 = round(_N16 * _COARSE)
_K2 = round(4 * _N16 * _COARSE + _N8 * _MEDIUM)


def _count_less(arrs, t):
    s = jnp.int32(0)
    for a in arrs:
        s = s + jnp.sum((a < t).astype(jnp.int32))
    return s


def _bisect_kth(arrs, k):
    lo = jnp.int32(0)
    for r in range(15):
        w = 1 << (28 - 2 * r)
        t1 = lo + w
        t2 = lo + 2 * w
        t3 = lo + 3 * w
        s1 = _count_less(arrs, t1)
        s2 = _count_less(arrs, t2)
        s3 = _count_less(arrs, t3)
        lo = jnp.where(k <= s1, lo,
                       jnp.where(k <= s2, t1,
                                 jnp.where(k <= s3, t2, t3)))
    return lo


def _select_body(x16_ref, x8_ref, oc_ref, om_ref):
    v16 = lax.bitcast_convert_type(x16_ref[...], jnp.int32)
    c_bits = _bisect_kth([v16], _K1)

    m16 = (v16 < c_bits).astype(jnp.bfloat16)

    x8v = x8_ref[...]
    row = lax.broadcasted_iota(jnp.int32, (128, 128), 0)
    col = lax.broadcasted_iota(jnp.int32, (128, 128), 1)
    masked = []
    for r in range(4):
        p_r = (row == (32 * r + (col % 64) // 2)).astype(jnp.bfloat16)
        mcols = jnp.dot(m16, p_r, preferred_element_type=jnp.float32)
        vals = jnp.where(mcols > 0.5, 0.0, x8v[:, 128 * r:128 * (r + 1)])
        masked.append(lax.bitcast_convert_type(vals, jnp.int32))
    m_bits = _bisect_kth(masked, _K2)

    oc_ref[0, 0] = c_bits
    om_ref[0, 0] = m_bits


def _expand(rows, cols):
    return (rows[:, None] == cols[None, :]).astype(jnp.bfloat16)


def _gates_body(x16_ref, x8_ref, ct_ref, mt_ref, gf_ref, gl_ref):
    cthr = ct_ref[0, 0]
    mthr = mt_ref[0, 0]
    x16 = x16_ref[...].reshape(256, 32)
    x8 = x8_ref[...].reshape(512, 64)

    i128 = lax.iota(jnp.int32, 128)
    i64 = lax.iota(jnp.int32, 64)
    i32 = lax.iota(jnp.int32, 32)

    gcf = (x16 < cthr).astype(jnp.bfloat16)

    c2c = _expand(i32, i64 // 2)
    gc2 = jnp.dot(jnp.repeat(gcf, 2, axis=0), c2c,
                  preferred_element_type=jnp.float32)

    gmf = ((x8 < mthr) & (gc2 < 0.5)).astype(jnp.bfloat16)

    c4c = _expand(i32, i128 // 4)
    gc4 = jnp.dot(jnp.repeat(gcf, 4, axis=0), c4c,
                  preferred_element_type=jnp.float32)

    c2f = _expand(i64, i128 // 2)
    gm2 = jnp.dot(jnp.repeat(gmf, 2, axis=0), c2f,
                  preferred_element_type=jnp.float32)

    gff = 1.0 - gc4 - gm2
    gf_ref[...] = gff.astype(jnp.int32).reshape(8, 128, 128)

    gl_ref[:, 0] = gc4.astype(jnp.int32).reshape(8, 128, 128)
    gl_ref[:, 1] = gm2.astype(jnp.int32).reshape(8, 128, 128)
    gl_ref[:, 2] = gff.astype(jnp.int32).reshape(8, 128, 128)


def _small_gates_body(x16t_ref, x8t_ref, ct_ref, mt_ref, gct_ref, gmt_ref):
    cthr = ct_ref[0, 0]
    mthr = mt_ref[0, 0]
    gct = (x16t_ref[...] < cthr).astype(jnp.int32)
    gct_ref[...] = gct
    m2 = jnp.repeat(jnp.repeat(gct, 2, axis=0), 2, axis=1)
    gmt_ref[...] = ((x8t_ref[...] < mthr) & (m2 == 0)).astype(jnp.int32)


@jax.jit
def kernel(x_entropy_p16, x_entropy_p8):
    x16f = x_entropy_p16.reshape(2048, 128)
    x8f = x_entropy_p8.reshape(2048, 512)

    c_bits, m_bits = pl.pallas_call(
        _select_body,
        out_specs=(pl.BlockSpec(memory_space=pltpu.SMEM),
                   pl.BlockSpec(memory_space=pltpu.SMEM)),
        out_shape=(jax.ShapeDtypeStruct((1, 1), jnp.int32),
                   jax.ShapeDtypeStruct((1, 1), jnp.int32)),
    )(x16f, x8f)
    cthr = lax.bitcast_convert_type(c_bits, jnp.float32)
    mthr = lax.bitcast_convert_type(m_bits, jnp.float32)

    gct, gmt = pl.pallas_call(
        _small_gates_body,
        in_specs=[
            pl.BlockSpec((32, 32, 256), lambda: (0, 0, 0)),
            pl.BlockSpec((64, 64, 256), lambda: (0, 0, 0)),
            pl.BlockSpec(memory_space=pltpu.SMEM),
            pl.BlockSpec(memory_space=pltpu.SMEM),
        ],
        out_shape=(
            jax.ShapeDtypeStruct((32, 32, 256), jnp.int32),
            jax.ShapeDtypeStruct((64, 64, 256), jnp.int32),
        ),
    )(x_entropy_p16.transpose(1, 2, 0), x_entropy_p8.transpose(1, 2, 0),
      cthr, mthr)

    grid = 32
    gf, gl = pl.pallas_call(
        _gates_body,
        grid=(grid,),
        in_specs=[
            pl.BlockSpec((8, 32, 32), lambda b: (b, 0, 0)),
            pl.BlockSpec((8, 64, 64), lambda b: (b, 0, 0)),
            pl.BlockSpec((1, 1), lambda b: (0, 0)),
            pl.BlockSpec((1, 1), lambda b: (0, 0)),
        ],
        out_specs=[
            pl.BlockSpec((8, 128, 128), lambda b: (b, 0, 0)),
            pl.BlockSpec((8, 3, 128, 128), lambda b: (b, 0, 0, 0)),
        ],
        out_shape=(
            jax.ShapeDtypeStruct((256, 128, 128), jnp.int32),
            jax.ShapeDtypeStruct((256, 3, 128, 128), jnp.int32),
        ),
    )(x_entropy_p16, x_entropy_p8, cthr, mthr)
    return (gct.transpose(2, 0, 1), gmt.transpose(2, 0, 1), gf,
            gl.transpose(0, 2, 3, 1))
